# Initial kernel scaffold; baseline (speedup 1.0000x reference)
import jax
import jax.numpy as jnp
from jax.experimental import pallas as pl


def kernel(omega, times, scores):
    raise NotImplementedError("write your pallas kernel here")


# speedup vs baseline: 8.1545x; 8.1545x over previous
import functools

import numpy as np
import jax
import jax.numpy as jnp
from jax import lax
from jax.experimental import pallas as pl
from jax.experimental.pallas import tpu as pltpu
from jax.experimental.pallas import tpu_sc as plsc

N = 1048576
DIM = 3
NT = 4096
OMEGA_BATCH = 1024
TIME_BATCH = 32
NLEV1 = 11
NLEV2 = 10
NTILES = 32
QPT = OMEGA_BATCH // NTILES


def _tf2x32(k0, k1, x0, x1):
    x0 = np.asarray(x0, np.uint32).copy()
    x1 = np.asarray(x1, np.uint32).copy()
    ks0 = np.uint32(k0)
    ks1 = np.uint32(k1)
    ks2 = np.uint32(ks0 ^ ks1 ^ np.uint32(0x1BD11BDA))
    rot = ([13, 15, 26, 6], [17, 29, 16, 24])
    ks = [ks0, ks1, ks2]
    x0 = (x0 + ks0).astype(np.uint32)
    x1 = (x1 + ks1).astype(np.uint32)
    for i in range(5):
        for r in rot[i % 2]:
            x0 = (x0 + x1).astype(np.uint32)
            x1 = ((x1 << np.uint32(r)) | (x1 >> np.uint32(32 - r))).astype(np.uint32)
            x1 = (x1 ^ x0).astype(np.uint32)
        x0 = (x0 + ks[(i + 1) % 3]).astype(np.uint32)
        x1 = (x1 + ks[(i + 2) % 3] + np.uint32(i + 1)).astype(np.uint32)
    return x0, x1


def _split2(k0, k1):
    b0, b1 = _tf2x32(k0, k1, np.zeros(2, np.uint32), np.arange(2, dtype=np.uint32))
    return (b0[0], b1[0]), (b0[1], b1[1])


def _random_bits(key, idx):
    idx = np.asarray(idx, np.uint32)
    o0, o1 = _tf2x32(key[0], key[1], np.zeros_like(idx), idx)
    return (o0 ^ o1).astype(np.uint32)


def _make_constants():
    k1, k2 = _split2(np.uint32(0), np.uint32(42))
    bits = _random_bits(k1, np.arange(OMEGA_BATCH))
    u = ((bits >> np.uint32(9)) | np.uint32(0x3F800000)).view(np.float32) - np.float32(1.0)
    u = np.maximum(np.float32(0.0), u)
    one_minus_u = (np.float32(1.0) - u).astype(np.float32)
    perm = np.arange(NT, dtype=np.int32)
    kk = k2
    for _ in range(2):
        kk, sub = _split2(kk[0], kk[1])
        perm = perm[np.argsort(_random_bits(sub, np.arange(NT)), kind="stable")]
    return one_minus_u, perm[:TIME_BATCH].astype(np.int32)


_ONE_MINUS_U, _T_IDX = _make_constants()


def _iota16():
    return jnp.arange(16, dtype=jnp.int32)


def _sc_body(p2d, s_arr, r_arr, tidx, times, omega_chunks, out2,
             s_v, r_v, b_v, low_v, high_v, wins, ind_v, gidx, obuf,
             tidx_v, times_v, tv_v, buf, sem):
    cc = lax.axis_index("c")
    ss = lax.axis_index("s")
    w = cc * 16 + ss
    base = w * QPT

    pltpu.sync_copy(s_arr, s_v)
    pltpu.sync_copy(r_arr.at[pl.ds(base, QPT)], r_v)
    pltpu.sync_copy(tidx, tidx_v)

    for v in range(2):
        rq = r_v[pl.ds(v * 16, 16)]
        lo = jnp.zeros((16,), jnp.int32)
        hi = jnp.full((16,), N, jnp.int32)

        def lvl1(_, lh, rq=rq):
            lo, hi = lh
            mid = lo + ((hi - lo) >> 1)
            sval = plsc.load_gather(s_v, [mid >> 9])
            go = rq <= sval
            return jnp.where(go, lo, mid), jnp.where(go, mid, hi)

        lo, hi = lax.fori_loop(0, NLEV1, lvl1, (lo, hi))
        b_v[pl.ds(v * 16, 16)] = lo >> 9
        low_v[pl.ds(v * 16, 16)] = lo
        high_v[pl.ds(v * 16, 16)] = hi

    pltpu.async_copy(p2d.at[b_v], wins, sem).wait()
    for v in range(2):
        rq = r_v[pl.ds(v * 16, 16)]
        lo = low_v[pl.ds(v * 16, 16)]
        hi = high_v[pl.ds(v * 16, 16)]
        lo0 = lo
        qrow = _iota16() + (v * 16)

        def lvl2(_, lh, rq=rq, lo0=lo0, qrow=qrow):
            lo, hi = lh
            mid = lo + ((hi - lo) >> 1)
            sval = plsc.load_gather(wins, [qrow, mid - lo0])
            go = rq <= sval
            return jnp.where(go, lo, mid), jnp.where(go, mid, hi)

        lo, hi = lax.fori_loop(0, NLEV2, lvl2, (lo, hi))
        ind_v[pl.ds(v * 16, 16)] = hi

    for v in range(4):
        j = _iota16() + v * 16
        q = j >> 1
        par = j & 1
        indq = plsc.load_gather(ind_v, [q])
        gidx[pl.ds(v * 16, 16)] = ((indq * 3) >> 7) + par
    pltpu.async_copy(omega_chunks.at[gidx], obuf, sem).wait()

    pltpu.sync_copy(times, times_v)
    for v in range(2):
        tq = _iota16() + v * 16
        tidxq = plsc.load_gather(tidx_v, [tq])
        tv_v[pl.ds(v * 16, 16)] = plsc.load_gather(times_v, [tidxq])

    for v in range(2):
        q = _iota16() + v * 16
        indq = plsc.load_gather(ind_v, [q])
        g0 = indq * 3
        chunk0 = g0 >> 7

        def _coord(d, indq=indq, g0=g0, chunk0=chunk0, q=q):
            g = g0 + d
            row = 2 * q + ((g >> 7) - chunk0)
            return plsc.load_gather(obuf, [row, g & 127])

        x0 = _coord(0)
        x1 = _coord(1)
        x2 = _coord(2)

        def tloop(t, carry, q=q, x0=x0, x1=x1, x2=x2):
            ts = jnp.full((16,), t, jnp.int32)
            col = q * 4
            tval = plsc.load_gather(tv_v, [ts])
            plsc.store_scatter(buf, [ts, col], tval)
            plsc.store_scatter(buf, [ts, col + 1], x0)
            plsc.store_scatter(buf, [ts, col + 2], x1)
            plsc.store_scatter(buf, [ts, col + 3], x2)
            return carry

        lax.fori_loop(0, TIME_BATCH, tloop, 0)

    pltpu.sync_copy(buf, out2.at[:, pl.ds(base * 4, QPT * 4)])


_sc_call = functools.partial(
    pl.kernel,
    out_type=jax.ShapeDtypeStruct((TIME_BATCH, OMEGA_BATCH * 4), jnp.float32),
    mesh=plsc.VectorSubcoreMesh(core_axis_name="c", subcore_axis_name="s"),
    compiler_params=pltpu.CompilerParams(needs_layout_passes=False),
    scratch_types=[
        pltpu.VMEM((2048,), jnp.float32),
        pltpu.VMEM((QPT,), jnp.float32),
        pltpu.VMEM((QPT,), jnp.int32),
        pltpu.VMEM((QPT,), jnp.int32),
        pltpu.VMEM((QPT,), jnp.int32),
        pltpu.VMEM((QPT, 512), jnp.float32),
        pltpu.VMEM((QPT,), jnp.int32),
        pltpu.VMEM((2 * QPT,), jnp.int32),
        pltpu.VMEM((2 * QPT, 128), jnp.float32),
        pltpu.VMEM((TIME_BATCH,), jnp.int32),
        pltpu.VMEM((NT,), jnp.float32),
        pltpu.VMEM((TIME_BATCH,), jnp.float32),
        pltpu.VMEM((TIME_BATCH, QPT * 4), jnp.float32),
        pltpu.SemaphoreType.DMA,
    ],
)(_sc_body)


def kernel(omega, times, scores):
    s = jnp.sum(scores)
    p = scores / s
    p_cuml = jnp.cumsum(p)
    r = p_cuml[-1] * jnp.asarray(_ONE_MINUS_U)
    p2d = p_cuml.reshape(N // 512, 512)
    s_arr = p_cuml[::512]
    tidx = jnp.asarray(_T_IDX)
    out2 = _sc_call(p2d, s_arr, r, tidx, times.reshape(NT),
                    omega.reshape(N * DIM // 128, 128))
    return out2.reshape(TIME_BATCH * OMEGA_BATCH, 4)


# speedup vs baseline: 43.4201x; 5.3247x over previous
import functools

import numpy as np
import jax
import jax.numpy as jnp
from jax import lax
from jax.experimental import pallas as pl
from jax.experimental.pallas import tpu as pltpu
from jax.experimental.pallas import tpu_sc as plsc

N = 1048576
DIM = 3
NT = 4096
OMEGA_BATCH = 1024
TIME_BATCH = 32
NLEV1 = 11
NLEV2 = 10
NTILES = 32
QPT = OMEGA_BATCH // NTILES


def _tf2x32(k0, k1, x0, x1):
    x0 = np.asarray(x0, np.uint32).copy()
    x1 = np.asarray(x1, np.uint32).copy()
    ks0 = np.uint32(k0)
    ks1 = np.uint32(k1)
    ks2 = np.uint32(ks0 ^ ks1 ^ np.uint32(0x1BD11BDA))
    rot = ([13, 15, 26, 6], [17, 29, 16, 24])
    ks = [ks0, ks1, ks2]
    x0 = (x0 + ks0).astype(np.uint32)
    x1 = (x1 + ks1).astype(np.uint32)
    for i in range(5):
        for r in rot[i % 2]:
            x0 = (x0 + x1).astype(np.uint32)
            x1 = ((x1 << np.uint32(r)) | (x1 >> np.uint32(32 - r))).astype(np.uint32)
            x1 = (x1 ^ x0).astype(np.uint32)
        x0 = (x0 + ks[(i + 1) % 3]).astype(np.uint32)
        x1 = (x1 + ks[(i + 2) % 3] + np.uint32(i + 1)).astype(np.uint32)
    return x0, x1


def _split2(k0, k1):
    b0, b1 = _tf2x32(k0, k1, np.zeros(2, np.uint32), np.arange(2, dtype=np.uint32))
    return (b0[0], b1[0]), (b0[1], b1[1])


def _random_bits(key, idx):
    idx = np.asarray(idx, np.uint32)
    o0, o1 = _tf2x32(key[0], key[1], np.zeros_like(idx), idx)
    return (o0 ^ o1).astype(np.uint32)


def _make_constants():
    k1, k2 = _split2(np.uint32(0), np.uint32(42))
    bits = _random_bits(k1, np.arange(OMEGA_BATCH))
    u = ((bits >> np.uint32(9)) | np.uint32(0x3F800000)).view(np.float32) - np.float32(1.0)
    u = np.maximum(np.float32(0.0), u)
    one_minus_u = (np.float32(1.0) - u).astype(np.float32)
    perm = np.arange(NT, dtype=np.int32)
    kk = k2
    for _ in range(2):
        kk, sub = _split2(kk[0], kk[1])
        perm = perm[np.argsort(_random_bits(sub, np.arange(NT)), kind="stable")]
    return one_minus_u, perm[:TIME_BATCH].astype(np.int32)


_ONE_MINUS_U, _T_IDX = _make_constants()


def _iota16():
    return jnp.arange(16, dtype=jnp.int32)


def _sc_body(p2d, s_arr, r_arr, tidx, times, omega_rows, out2,
             s_v, r_v, b_v, low_v, high_v, wins, ind_v, gidx, obuf,
             tidx_v, times_v, tv_v, buf, sem):
    cc = lax.axis_index("c")
    ss = lax.axis_index("s")
    w = cc * 16 + ss
    base = w * QPT

    pltpu.sync_copy(s_arr, s_v)
    pltpu.sync_copy(r_arr.at[pl.ds(base, QPT)], r_v)
    pltpu.sync_copy(tidx, tidx_v)

    for v in range(2):
        rq = r_v[pl.ds(v * 16, 16)]
        lo = jnp.zeros((16,), jnp.int32)
        hi = jnp.full((16,), N, jnp.int32)

        def lvl1(_, lh, rq=rq):
            lo, hi = lh
            mid = lo + ((hi - lo) >> 1)
            sval = plsc.load_gather(s_v, [mid >> 9])
            go = rq <= sval
            return jnp.where(go, lo, mid), jnp.where(go, mid, hi)

        lo, hi = lax.fori_loop(0, NLEV1, lvl1, (lo, hi))
        b_v[pl.ds(v * 16, 16)] = lo >> 9
        low_v[pl.ds(v * 16, 16)] = lo
        high_v[pl.ds(v * 16, 16)] = hi

    pltpu.async_copy(p2d.at[b_v], wins, sem).wait()
    for v in range(2):
        rq = r_v[pl.ds(v * 16, 16)]
        lo = low_v[pl.ds(v * 16, 16)]
        hi = high_v[pl.ds(v * 16, 16)]
        lo0 = lo
        qrow = _iota16() + (v * 16)

        def lvl2(_, lh, rq=rq, lo0=lo0, qrow=qrow):
            lo, hi = lh
            mid = lo + ((hi - lo) >> 1)
            sval = plsc.load_gather(wins, [qrow, mid - lo0])
            go = rq <= sval
            return jnp.where(go, lo, mid), jnp.where(go, mid, hi)

        lo, hi = lax.fori_loop(0, NLEV2, lvl2, (lo, hi))
        ind_v[pl.ds(v * 16, 16)] = hi

    for v in range(6):
        q = _iota16() + ((v & 1) * 16)
        indq = plsc.load_gather(ind_v, [q])
        gidx[pl.ds(v * 16, 16)] = (v >> 1) * (N // 128) + (indq >> 7)
    pltpu.async_copy(omega_rows.at[gidx], obuf, sem).wait()

    pltpu.sync_copy(times, times_v)
    for v in range(2):
        tq = _iota16() + v * 16
        tidxq = plsc.load_gather(tidx_v, [tq])
        tv_v[pl.ds(v * 16, 16)] = plsc.load_gather(times_v, [tidxq])

    for v in range(2):
        q = _iota16() + v * 16
        indq = plsc.load_gather(ind_v, [q])
        col = indq & 127

        def _coord(d, col=col, q=q):
            return plsc.load_gather(obuf, [q + d * QPT, col])

        x0 = _coord(0)
        x1 = _coord(1)
        x2 = _coord(2)

        def tloop(t, carry, q=q, x0=x0, x1=x1, x2=x2):
            ts = jnp.full((16,), t, jnp.int32)
            col = q * 4
            tval = plsc.load_gather(tv_v, [ts])
            plsc.store_scatter(buf, [ts, col], tval)
            plsc.store_scatter(buf, [ts, col + 1], x0)
            plsc.store_scatter(buf, [ts, col + 2], x1)
            plsc.store_scatter(buf, [ts, col + 3], x2)
            return carry

        lax.fori_loop(0, TIME_BATCH, tloop, 0)

    pltpu.sync_copy(buf, out2.at[:, pl.ds(base * 4, QPT * 4)])


_sc_call = functools.partial(
    pl.kernel,
    out_type=jax.ShapeDtypeStruct((TIME_BATCH, OMEGA_BATCH * 4), jnp.float32),
    mesh=plsc.VectorSubcoreMesh(core_axis_name="c", subcore_axis_name="s"),
    compiler_params=pltpu.CompilerParams(needs_layout_passes=False),
    scratch_types=[
        pltpu.VMEM((2048,), jnp.float32),
        pltpu.VMEM((QPT,), jnp.float32),
        pltpu.VMEM((QPT,), jnp.int32),
        pltpu.VMEM((QPT,), jnp.int32),
        pltpu.VMEM((QPT,), jnp.int32),
        pltpu.VMEM((QPT, 512), jnp.float32),
        pltpu.VMEM((QPT,), jnp.int32),
        pltpu.VMEM((3 * QPT,), jnp.int32),
        pltpu.VMEM((3 * QPT, 128), jnp.float32),
        pltpu.VMEM((TIME_BATCH,), jnp.int32),
        pltpu.VMEM((NT,), jnp.float32),
        pltpu.VMEM((TIME_BATCH,), jnp.float32),
        pltpu.VMEM((TIME_BATCH, QPT * 4), jnp.float32),
        pltpu.SemaphoreType.DMA,
    ],
)(_sc_body)


def kernel(omega, times, scores):
    s = jnp.sum(scores)
    p = scores / s
    p_cuml = jnp.cumsum(p)
    r = p_cuml[-1] * jnp.asarray(_ONE_MINUS_U)
    p2d = p_cuml.reshape(N // 512, 512)
    s_arr = p_cuml[::512]
    tidx = jnp.asarray(_T_IDX)
    out2 = _sc_call(p2d, s_arr, r, tidx, times.reshape(NT),
                    omega.T.reshape(DIM * (N // 128), 128))
    return out2.reshape(TIME_BATCH * OMEGA_BATCH, 4)
